# Initial kernel scaffold; baseline (speedup 1.0000x reference)
import jax
import jax.numpy as jnp
from jax.experimental import pallas as pl


def kernel(ui_row, ui_col, ub_row, ub_col, bi_row, bi_col, users_feat, items_feat, bundles_feat):
    raise NotImplementedError("write your pallas kernel here")


# speedup vs baseline: 3.1568x; 3.1568x over previous
import functools

import jax
import jax.numpy as jnp
from jax import lax
from jax.experimental import pallas as pl
from jax.experimental.pallas import tpu as pltpu
from jax.experimental.pallas import tpu_sc as plsc

_U, _I, _BN, _D = 100000, 50000, 20000, 64
_EPS = 1e-08

_NC, _NS, _L = 2, 16, 16
_BATCH = 128
_NBATCH = 16
_CHUNK = _BATCH * _NBATCH
_EDGE_ALIGN = _NS * _CHUNK
_BLOCK_ROWS = 24576
_ZB = 256
_ZBH = 512

_SC_PARAMS = pltpu.CompilerParams(use_tc_tiling_on_sc=False)


def _cdiv(a, b):
    return -(-a // b)


def _make_spmm(e_rows, num_blocks, n_g):
    acc_rows = _BLOCK_ROWS + _BATCH
    bpc = _cdiv(num_blocks, _NC)
    ztot = acc_rows // _NS
    zfull, zrem = ztot // _ZB, ztot % _ZB
    frows = _BLOCK_ROWS // _NS
    rows_per_tile = e_rows // _NS
    nchunk = rows_per_tile // _NBATCH
    mesh = plsc.VectorSubcoreMesh(core_axis_name="c", subcore_axis_name="s")

    @functools.partial(
        pl.kernel,
        out_type=jax.ShapeDtypeStruct((num_blocks * _BLOCK_ROWS, _D),
                                      jnp.float32),
        mesh=mesh,
        scratch_types=[
            pltpu.VMEM_SHARED((acc_rows, _D), jnp.float32),
            pltpu.VMEM((_NBATCH, _BATCH), jnp.int32),
            pltpu.VMEM((_NBATCH, _BATCH), jnp.int32),
            pltpu.VMEM((_NBATCH, _BATCH), jnp.int32),
            pltpu.VMEM((_BATCH, _D), jnp.float32),
            pltpu.VMEM((_ZB, _D), jnp.float32),
        ],
        compiler_params=_SC_PARAMS,
    )
    def spmm(r_hbm, c_hbm, g_hbm, z_hbm, out_hbm,
             acc, rbuf, cbuf, ibuf, gbuf, zbuf):
        core = lax.axis_index("c")
        sid = lax.axis_index("s")
        pltpu.sync_copy(z_hbm, zbuf)
        for bi in range(bpc):
            blk = core * bpc + bi

            @pl.when(blk < num_blocks)
            def _():
                lo = blk * _BLOCK_ROWS
                zbase = sid * ztot
                for k in range(zfull):
                    pltpu.sync_copy(zbuf, acc.at[pl.ds(zbase + k * _ZB, _ZB)])
                if zrem:
                    pltpu.sync_copy(zbuf.at[pl.ds(0, zrem)],
                                    acc.at[pl.ds(zbase + zfull * _ZB, zrem)])
                plsc.subcore_barrier()

                @pl.loop(0, nchunk)
                def _(ch):
                    rowbase = sid * rows_per_tile + ch * _NBATCH
                    pltpu.sync_copy(r_hbm.at[pl.ds(rowbase, _NBATCH)], rbuf)
                    pltpu.sync_copy(c_hbm.at[pl.ds(rowbase, _NBATCH)], cbuf)

                    @pl.loop(0, _NBATCH)
                    def _(b):
                        @pl.loop(0, _BATCH // _L)
                        def _(kk):
                            rv = rbuf[b, pl.ds(kk * _L, _L)]
                            lv = rv - lo
                            ok = (lv >= 0) & (lv < _BLOCK_ROWS)
                            ibuf[b, pl.ds(kk * _L, _L)] = jnp.where(
                                ok, lv, _BLOCK_ROWS)

                    @pl.loop(0, _NBATCH)
                    def _(b):
                        pltpu.sync_copy(g_hbm.at[cbuf.at[b]], gbuf)
                        pltpu.sync_copy(gbuf, acc.at[ibuf.at[b]], add=True)

                plsc.subcore_barrier()
                pltpu.sync_copy(
                    acc.at[pl.ds(sid * frows, frows)],
                    out_hbm.at[pl.ds(lo + sid * frows, frows)])
                plsc.subcore_barrier()

    return spmm


def _make_hist(e_rows, num_blocks, block_rows):
    acc_rows = block_rows + _BATCH
    bpc = _cdiv(num_blocks, _NC)
    ztot = acc_rows // _NS
    zfull, zrem = ztot // _ZBH, ztot % _ZBH
    frows = block_rows // _NS
    rows_per_tile = e_rows // _NS
    nchunk = rows_per_tile // _NBATCH
    mesh = plsc.VectorSubcoreMesh(core_axis_name="c", subcore_axis_name="s")

    @functools.partial(
        pl.kernel,
        out_type=jax.ShapeDtypeStruct((num_blocks * block_rows, _L),
                                      jnp.float32),
        mesh=mesh,
        scratch_types=[
            pltpu.VMEM_SHARED((acc_rows, _L), jnp.float32),
            pltpu.VMEM((_NBATCH, _BATCH), jnp.int32),
            pltpu.VMEM((_NBATCH, _BATCH), jnp.int32),
            pltpu.VMEM((_BATCH, _L), jnp.float32),
            pltpu.VMEM((_ZBH, _L), jnp.float32),
        ],
        compiler_params=_SC_PARAMS,
    )
    def hist(r_hbm, ones_hbm, z_hbm, out_hbm, acc, rbuf, ibuf, obuf, zbuf):
        core = lax.axis_index("c")
        sid = lax.axis_index("s")
        pltpu.sync_copy(z_hbm, zbuf)
        pltpu.sync_copy(ones_hbm, obuf)
        for bi in range(bpc):
            blk = core * bpc + bi

            @pl.when(blk < num_blocks)
            def _():
                lo = blk * block_rows
                zbase = sid * ztot
                for k in range(zfull):
                    pltpu.sync_copy(zbuf, acc.at[pl.ds(zbase + k * _ZBH, _ZBH)])
                if zrem:
                    pltpu.sync_copy(zbuf.at[pl.ds(0, zrem)],
                                    acc.at[pl.ds(zbase + zfull * _ZBH, zrem)])
                plsc.subcore_barrier()

                @pl.loop(0, nchunk)
                def _(ch):
                    rowbase = sid * rows_per_tile + ch * _NBATCH
                    pltpu.sync_copy(r_hbm.at[pl.ds(rowbase, _NBATCH)], rbuf)

                    @pl.loop(0, _NBATCH)
                    def _(b):
                        @pl.loop(0, _BATCH // _L)
                        def _(kk):
                            rv = rbuf[b, pl.ds(kk * _L, _L)]
                            lv = rv - lo
                            ok = (lv >= 0) & (lv < block_rows)
                            ibuf[b, pl.ds(kk * _L, _L)] = jnp.where(
                                ok, lv, block_rows)

                    @pl.loop(0, _NBATCH)
                    def _(b):
                        pltpu.sync_copy(obuf, acc.at[ibuf.at[b]], add=True)

                plsc.subcore_barrier()
                pltpu.sync_copy(
                    acc.at[pl.ds(sid * frows, frows)],
                    out_hbm.at[pl.ds(lo + sid * frows, frows)])
                plsc.subcore_barrier()

    return hist


_BM = 512


def _prescale_body(deg_ref, f_ref, o_ref):
    fs = 1.0 / (jnp.sqrt(deg_ref[...]) + _EPS)
    o_ref[...] = f_ref[...] * fs[:, None]


def _mid_body(deg_ref, t_ref, o_ref):
    fs = 1.0 / (jnp.sqrt(deg_ref[...]) + _EPS)
    o_ref[...] = t_ref[...] * (0.5 * fs * fs)[:, None]


def _l2n(x):
    nrm = jnp.sqrt(jnp.sum(x * x, axis=1, keepdims=True))
    return x / jnp.maximum(nrm, 1e-12)


def _final_body(f_ref, t1_ref, t2_ref, o_ref):
    o_ref[...] = f_ref[...] + _l2n(t1_ref[...]) + _l2n(t2_ref[...])


def _bscale_body(sz_ref, t_ref, o_ref):
    o_ref[...] = t_ref[...] / (sz_ref[...] + 1e-08)[:, None]


def _ew_call(body, n, ins):
    grid = (n // _BM,)
    in_specs = []
    for x in ins:
        if x.ndim == 1:
            in_specs.append(pl.BlockSpec((_BM,), lambda i: (i,)))
        else:
            in_specs.append(pl.BlockSpec((_BM, _D), lambda i: (i, 0)))
    return pl.pallas_call(
        body,
        grid=grid,
        in_specs=in_specs,
        out_specs=pl.BlockSpec((_BM, _D), lambda i: (i, 0)),
        out_shape=jax.ShapeDtypeStruct((n, _D), jnp.float32),
    )(*ins)


def _pad_rows(x, m, value=0):
    n = x.shape[0]
    npad = _cdiv(n, m) * m - n
    if npad == 0:
        return x
    cfg = ((0, npad),) + ((0, 0),) * (x.ndim - 1)
    return jnp.pad(x, cfg, constant_values=value)


def _edge_arrays(r, c, n):
    r_p = _pad_rows(r.astype(jnp.int32), _EDGE_ALIGN, value=n)
    c_p = _pad_rows(c.astype(jnp.int32), _EDGE_ALIGN, value=0)
    return r_p.reshape(-1, _BATCH), c_p.reshape(-1, _BATCH)


def _hist_call(r2d, n):
    block_rows = _cdiv(_cdiv(n, _NC), _L) * _L
    num_blocks = _cdiv(n, block_rows)
    hist = _make_hist(r2d.shape[0], num_blocks, block_rows)
    ones = jnp.ones((_BATCH, _L), jnp.float32)
    zeros = jnp.zeros((_ZBH, _L), jnp.float32)
    out = hist(r2d, ones, zeros)
    return out[:n, 0]


def _propagate(r, c, n, feats):
    r2d, c2d = _edge_arrays(r, c, n)
    deg = _hist_call(r2d, n)
    num_blocks = _cdiv(n, _BLOCK_ROWS)
    np_rows = num_blocks * _BLOCK_ROWS
    deg_p = _pad_rows(deg, np_rows)
    feats_p = _pad_rows(feats, np_rows)
    spmm = _make_spmm(r2d.shape[0], num_blocks, np_rows)
    zeros = jnp.zeros((_ZB, _D), jnp.float32)
    g1 = _ew_call(_prescale_body, np_rows, [deg_p, feats_p])
    t1 = spmm(r2d, c2d, g1, zeros)
    g2 = _ew_call(_mid_body, np_rows, [deg_p, t1])
    t2 = spmm(r2d, c2d, g2, zeros)
    acc = _ew_call(_final_body, np_rows, [feats_p, t1, t2])
    return acc[:n]


def _bundle_agg(bi_row, bi_col, aff_items):
    n = _BN
    r2d, c2d = _edge_arrays(bi_row, bi_col, n)
    sz = _hist_call(r2d, n)
    num_blocks = _cdiv(n, _BLOCK_ROWS)
    np_rows = num_blocks * _BLOCK_ROWS
    sz_p = _pad_rows(sz, np_rows)[:np_rows]
    spmm = _make_spmm(r2d.shape[0], num_blocks, aff_items.shape[0])
    zeros = jnp.zeros((_ZB, _D), jnp.float32)
    t = spmm(r2d, c2d, aff_items, zeros)
    out = _ew_call(_bscale_body, np_rows, [sz_p, t])
    return out[:n]


def kernel(ui_row, ui_col, ub_row, ub_col, bi_row, bi_col,
           users_feat, items_feat, bundles_feat):
    r1 = jnp.concatenate([ui_row, ui_col + _U])
    c1 = jnp.concatenate([ui_col + _U, ui_row])
    acc1 = _propagate(r1, c1, _U + _I,
                      jnp.concatenate([users_feat, items_feat], axis=0))
    aff_users, aff_items = acc1[:_U], acc1[_U:]

    r2 = jnp.concatenate([ub_row, ub_col + _U])
    c2 = jnp.concatenate([ub_col + _U, ub_row])
    acc2 = _propagate(r2, c2, _U + _BN,
                      jnp.concatenate([users_feat, bundles_feat], axis=0))
    hist_users, hist_bundles = acc2[:_U], acc2[_U:]

    aff_bundles = _bundle_agg(bi_row, bi_col, aff_items)
    return jnp.concatenate(
        [aff_users, hist_users, aff_bundles, hist_bundles], axis=0)


# speedup vs baseline: 3.2165x; 1.0189x over previous
import functools

import jax
import jax.numpy as jnp
from jax import lax
from jax.experimental import pallas as pl
from jax.experimental.pallas import tpu as pltpu
from jax.experimental.pallas import tpu_sc as plsc

_U, _I, _BN, _D = 100000, 50000, 20000, 64
_EPS = 1e-08

_NC, _NS, _L = 2, 16, 16
_BATCH = 128
_NBATCH = 16
_CHUNK = _BATCH * _NBATCH
_EDGE_ALIGN = _NS * _CHUNK
_BLOCK_ROWS = 24576
_ZB = 128
_ZBH = 512

_SC_PARAMS = pltpu.CompilerParams(use_tc_tiling_on_sc=False)


def _cdiv(a, b):
    return -(-a // b)


def _make_spmm(e_rows, num_blocks, n_g):
    acc_rows = _BLOCK_ROWS + _BATCH
    bpc = _cdiv(num_blocks, _NC)
    ztot = acc_rows // _NS
    zfull, zrem = ztot // _ZB, ztot % _ZB
    frows = _BLOCK_ROWS // _NS
    rows_per_tile = e_rows // _NS
    nchunk = rows_per_tile // _NBATCH
    mesh = plsc.VectorSubcoreMesh(core_axis_name="c", subcore_axis_name="s")

    @functools.partial(
        pl.kernel,
        out_type=jax.ShapeDtypeStruct((num_blocks * _BLOCK_ROWS, _D),
                                      jnp.float32),
        mesh=mesh,
        scratch_types=[
            pltpu.VMEM_SHARED((acc_rows, _D), jnp.float32),
            pltpu.VMEM((_NBATCH, _BATCH), jnp.int32),
            pltpu.VMEM((_NBATCH, _BATCH), jnp.int32),
            pltpu.VMEM((_NBATCH, _BATCH), jnp.int32),
            pltpu.VMEM((_BATCH, _D), jnp.float32),
            pltpu.VMEM((_BATCH, _D), jnp.float32),
            pltpu.VMEM((_ZB, _D), jnp.float32),
            pltpu.SemaphoreType.DMA,
            pltpu.SemaphoreType.DMA,
            pltpu.SemaphoreType.DMA,
            pltpu.SemaphoreType.DMA,
        ],
        compiler_params=_SC_PARAMS,
    )
    def spmm(r_hbm, c_hbm, g_hbm, z_hbm, out_hbm,
             acc, rbuf, cbuf, ibuf, gbuf0, gbuf1, zbuf,
             gsem0, gsem1, ssem0, ssem1):
        core = lax.axis_index("c")
        sid = lax.axis_index("s")
        pltpu.sync_copy(z_hbm, zbuf)
        for bi in range(bpc):
            blk = core * bpc + bi

            @pl.when(blk < num_blocks)
            def _():
                lo = blk * _BLOCK_ROWS
                zbase = sid * ztot
                for k in range(zfull):
                    pltpu.sync_copy(zbuf, acc.at[pl.ds(zbase + k * _ZB, _ZB)])
                if zrem:
                    pltpu.sync_copy(zbuf.at[pl.ds(0, zrem)],
                                    acc.at[pl.ds(zbase + zfull * _ZB, zrem)])
                plsc.subcore_barrier()

                @pl.loop(0, nchunk)
                def _(ch):
                    rowbase = sid * rows_per_tile + ch * _NBATCH
                    pltpu.sync_copy(r_hbm.at[pl.ds(rowbase, _NBATCH)], rbuf)
                    pltpu.sync_copy(c_hbm.at[pl.ds(rowbase, _NBATCH)], cbuf)

                    @pl.loop(0, _NBATCH)
                    def _(b):
                        @pl.loop(0, _BATCH // _L)
                        def _(kk):
                            rv = rbuf[b, pl.ds(kk * _L, _L)]
                            lv = rv - lo
                            ok = (lv >= 0) & (lv < _BLOCK_ROWS)
                            ibuf[b, pl.ds(kk * _L, _L)] = jnp.where(
                                ok, lv, _BLOCK_ROWS)

                    gbufs = (gbuf0, gbuf1)
                    gsems = (gsem0, gsem1)
                    ssems = (ssem0, ssem1)
                    pltpu.async_copy(g_hbm.at[cbuf.at[0]], gbuf0, gsem0)
                    for b in range(_NBATCH):
                        p = b % 2
                        q = (b + 1) % 2
                        pltpu.make_async_copy(
                            g_hbm.at[cbuf.at[b]], gbufs[p], gsems[p]).wait()
                        pltpu.async_copy(gbufs[p], acc.at[ibuf.at[b]],
                                         ssems[p], add=True)
                        if b + 1 < _NBATCH:
                            if b >= 1:
                                pltpu.make_async_copy(
                                    gbufs[q], acc.at[ibuf.at[b - 1]],
                                    ssems[q]).wait()
                            pltpu.async_copy(g_hbm.at[cbuf.at[b + 1]],
                                             gbufs[q], gsems[q])
                    pltpu.make_async_copy(
                        gbufs[(_NBATCH - 2) % 2],
                        acc.at[ibuf.at[_NBATCH - 2]],
                        ssems[(_NBATCH - 2) % 2]).wait()
                    pltpu.make_async_copy(
                        gbufs[(_NBATCH - 1) % 2],
                        acc.at[ibuf.at[_NBATCH - 1]],
                        ssems[(_NBATCH - 1) % 2]).wait()

                plsc.subcore_barrier()
                pltpu.sync_copy(
                    acc.at[pl.ds(sid * frows, frows)],
                    out_hbm.at[pl.ds(lo + sid * frows, frows)])
                plsc.subcore_barrier()

    return spmm


def _make_hist(e_rows, num_blocks, block_rows):
    acc_rows = block_rows + _BATCH
    bpc = _cdiv(num_blocks, _NC)
    ztot = acc_rows // _NS
    zfull, zrem = ztot // _ZBH, ztot % _ZBH
    frows = block_rows // _NS
    rows_per_tile = e_rows // _NS
    nchunk = rows_per_tile // _NBATCH
    mesh = plsc.VectorSubcoreMesh(core_axis_name="c", subcore_axis_name="s")

    @functools.partial(
        pl.kernel,
        out_type=jax.ShapeDtypeStruct((num_blocks * block_rows, _L),
                                      jnp.float32),
        mesh=mesh,
        scratch_types=[
            pltpu.VMEM_SHARED((acc_rows, _L), jnp.float32),
            pltpu.VMEM((_NBATCH, _BATCH), jnp.int32),
            pltpu.VMEM((_NBATCH, _BATCH), jnp.int32),
            pltpu.VMEM((_BATCH, _L), jnp.float32),
            pltpu.VMEM((_ZBH, _L), jnp.float32),
            pltpu.SemaphoreType.DMA,
        ],
        compiler_params=_SC_PARAMS,
    )
    def hist(r_hbm, ones_hbm, z_hbm, out_hbm, acc, rbuf, ibuf, obuf, zbuf,
             hsem):
        core = lax.axis_index("c")
        sid = lax.axis_index("s")
        pltpu.sync_copy(z_hbm, zbuf)
        pltpu.sync_copy(ones_hbm, obuf)
        for bi in range(bpc):
            blk = core * bpc + bi

            @pl.when(blk < num_blocks)
            def _():
                lo = blk * block_rows
                zbase = sid * ztot
                for k in range(zfull):
                    pltpu.sync_copy(zbuf, acc.at[pl.ds(zbase + k * _ZBH, _ZBH)])
                if zrem:
                    pltpu.sync_copy(zbuf.at[pl.ds(0, zrem)],
                                    acc.at[pl.ds(zbase + zfull * _ZBH, zrem)])
                plsc.subcore_barrier()

                @pl.loop(0, nchunk)
                def _(ch):
                    rowbase = sid * rows_per_tile + ch * _NBATCH
                    pltpu.sync_copy(r_hbm.at[pl.ds(rowbase, _NBATCH)], rbuf)

                    @pl.loop(0, _NBATCH)
                    def _(b):
                        @pl.loop(0, _BATCH // _L)
                        def _(kk):
                            rv = rbuf[b, pl.ds(kk * _L, _L)]
                            lv = rv - lo
                            ok = (lv >= 0) & (lv < block_rows)
                            ibuf[b, pl.ds(kk * _L, _L)] = jnp.where(
                                ok, lv, block_rows)

                    @pl.loop(0, _NBATCH)
                    def _(b):
                        pltpu.async_copy(obuf, acc.at[ibuf.at[b]], hsem,
                                         add=True)

                    @pl.loop(0, _NBATCH)
                    def _(b):
                        pltpu.make_async_copy(obuf, acc.at[ibuf.at[b]],
                                              hsem).wait()

                plsc.subcore_barrier()
                pltpu.sync_copy(
                    acc.at[pl.ds(sid * frows, frows)],
                    out_hbm.at[pl.ds(lo + sid * frows, frows)])
                plsc.subcore_barrier()

    return hist


_BM = 512


def _prescale_body(deg_ref, f_ref, o_ref):
    fs = 1.0 / (jnp.sqrt(deg_ref[...]) + _EPS)
    o_ref[...] = f_ref[...] * fs[:, None]


def _mid_body(deg_ref, t_ref, o_ref):
    fs = 1.0 / (jnp.sqrt(deg_ref[...]) + _EPS)
    o_ref[...] = t_ref[...] * (0.5 * fs * fs)[:, None]


def _l2n(x):
    nrm = jnp.sqrt(jnp.sum(x * x, axis=1, keepdims=True))
    return x / jnp.maximum(nrm, 1e-12)


def _final_body(f_ref, t1_ref, t2_ref, o_ref):
    o_ref[...] = f_ref[...] + _l2n(t1_ref[...]) + _l2n(t2_ref[...])


def _bscale_body(sz_ref, t_ref, o_ref):
    o_ref[...] = t_ref[...] / (sz_ref[...] + 1e-08)[:, None]


def _ew_call(body, n, ins):
    grid = (n // _BM,)
    in_specs = []
    for x in ins:
        if x.ndim == 1:
            in_specs.append(pl.BlockSpec((_BM,), lambda i: (i,)))
        else:
            in_specs.append(pl.BlockSpec((_BM, _D), lambda i: (i, 0)))
    return pl.pallas_call(
        body,
        grid=grid,
        in_specs=in_specs,
        out_specs=pl.BlockSpec((_BM, _D), lambda i: (i, 0)),
        out_shape=jax.ShapeDtypeStruct((n, _D), jnp.float32),
    )(*ins)


def _pad_rows(x, m, value=0):
    n = x.shape[0]
    npad = _cdiv(n, m) * m - n
    if npad == 0:
        return x
    cfg = ((0, npad),) + ((0, 0),) * (x.ndim - 1)
    return jnp.pad(x, cfg, constant_values=value)


def _edge_arrays(r, c, n):
    r_p = _pad_rows(r.astype(jnp.int32), _EDGE_ALIGN, value=n)
    c_p = _pad_rows(c.astype(jnp.int32), _EDGE_ALIGN, value=0)
    return r_p.reshape(-1, _BATCH), c_p.reshape(-1, _BATCH)


def _hist_call(r2d, n):
    block_rows = _cdiv(_cdiv(n, _NC), _L) * _L
    num_blocks = _cdiv(n, block_rows)
    hist = _make_hist(r2d.shape[0], num_blocks, block_rows)
    ones = jnp.ones((_BATCH, _L), jnp.float32)
    zeros = jnp.zeros((_ZBH, _L), jnp.float32)
    out = hist(r2d, ones, zeros)
    return out[:n, 0]


def _propagate(r, c, n, feats):
    r2d, c2d = _edge_arrays(r, c, n)
    deg = _hist_call(r2d, n)
    num_blocks = _cdiv(n, _BLOCK_ROWS)
    np_rows = num_blocks * _BLOCK_ROWS
    deg_p = _pad_rows(deg, np_rows)
    feats_p = _pad_rows(feats, np_rows)
    spmm = _make_spmm(r2d.shape[0], num_blocks, np_rows)
    zeros = jnp.zeros((_ZB, _D), jnp.float32)
    g1 = _ew_call(_prescale_body, np_rows, [deg_p, feats_p])
    t1 = spmm(r2d, c2d, g1, zeros)
    g2 = _ew_call(_mid_body, np_rows, [deg_p, t1])
    t2 = spmm(r2d, c2d, g2, zeros)
    acc = _ew_call(_final_body, np_rows, [feats_p, t1, t2])
    return acc[:n]


def _bundle_agg(bi_row, bi_col, aff_items):
    n = _BN
    r2d, c2d = _edge_arrays(bi_row, bi_col, n)
    sz = _hist_call(r2d, n)
    num_blocks = _cdiv(n, _BLOCK_ROWS)
    np_rows = num_blocks * _BLOCK_ROWS
    sz_p = _pad_rows(sz, np_rows)[:np_rows]
    spmm = _make_spmm(r2d.shape[0], num_blocks, aff_items.shape[0])
    zeros = jnp.zeros((_ZB, _D), jnp.float32)
    t = spmm(r2d, c2d, aff_items, zeros)
    out = _ew_call(_bscale_body, np_rows, [sz_p, t])
    return out[:n]


def kernel(ui_row, ui_col, ub_row, ub_col, bi_row, bi_col,
           users_feat, items_feat, bundles_feat):
    r1 = jnp.concatenate([ui_row, ui_col + _U])
    c1 = jnp.concatenate([ui_col + _U, ui_row])
    acc1 = _propagate(r1, c1, _U + _I,
                      jnp.concatenate([users_feat, items_feat], axis=0))
    aff_users, aff_items = acc1[:_U], acc1[_U:]

    r2 = jnp.concatenate([ub_row, ub_col + _U])
    c2 = jnp.concatenate([ub_col + _U, ub_row])
    acc2 = _propagate(r2, c2, _U + _BN,
                      jnp.concatenate([users_feat, bundles_feat], axis=0))
    hist_users, hist_bundles = acc2[:_U], acc2[_U:]

    aff_bundles = _bundle_agg(bi_row, bi_col, aff_items)
    return jnp.concatenate(
        [aff_users, hist_users, aff_bundles, hist_bundles], axis=0)
